# Initial kernel scaffold; baseline (speedup 1.0000x reference)
import jax
import jax.numpy as jnp
from jax import lax
from jax.experimental import pallas as pl
from jax.experimental.pallas import tpu as pltpu
from jax.experimental.pallas import tpu_sc as plsc

N = 10000
D = 128
NC = 2
NS = 16
NW = NC * NS
CHUNK = 128
N_PAD = 10240
E = 320000
TPC = 79
EPT = TPC * CHUNK
E_PAD = NW * EPT
SEG = N_PAD // NS

_MESH = plsc.VectorSubcoreMesh(
    core_axis_name="c", subcore_axis_name="s", num_cores=NC, num_subcores=NS
)


def _deg_body(src_hbm, dst_hbm, z2_hbm, out_hbm, idx_v, ones_v, dout_acc, din_acc):
    c = lax.axis_index("c")
    s = lax.axis_index("s")
    wid = s * NC + c
    pltpu.sync_copy(z2_hbm.at[0, pl.ds(s * SEG, SEG)], dout_acc.at[pl.ds(s * SEG, SEG)])
    pltpu.sync_copy(z2_hbm.at[1, pl.ds(s * SEG, SEG)], din_acc.at[pl.ds(s * SEG, SEG)])

    def _ones(i, carry):
        ones_v[pl.ds(i * 16, 16)] = jnp.ones((16,), jnp.float32)
        return carry

    lax.fori_loop(0, CHUNK // 16, _ones, 0)
    plsc.subcore_barrier()

    def _step(j, carry):
        pltpu.sync_copy(src_hbm.at[wid, j], idx_v)
        pltpu.sync_copy(ones_v, dout_acc.at[idx_v], add=True)
        pltpu.sync_copy(dst_hbm.at[wid, j], idx_v)
        pltpu.sync_copy(ones_v, din_acc.at[idx_v], add=True)
        return carry

    lax.fori_loop(0, TPC, _step, 0)
    plsc.subcore_barrier()
    pltpu.sync_copy(dout_acc.at[pl.ds(s * SEG, SEG)], out_hbm.at[c, 0, pl.ds(s * SEG, SEG)])
    pltpu.sync_copy(din_acc.at[pl.ds(s * SEG, SEG)], out_hbm.at[c, 1, pl.ds(s * SEG, SEG)])


_deg = pl.kernel(
    _deg_body,
    out_type=jax.ShapeDtypeStruct((NC, 2, N_PAD), jnp.float32),
    mesh=_MESH,
    scratch_types=[
        pltpu.VMEM((CHUNK,), jnp.int32),
        pltpu.VMEM((CHUNK,), jnp.float32),
        pltpu.VMEM_SHARED((N_PAD,), jnp.float32),
        pltpu.VMEM_SHARED((N_PAD,), jnp.float32),
    ],
)


def _agg_body(src_hbm, dst_hbm, hs_hbm, z_hbm, out_hbm, sidx, didx, rows, acc, sem):
    c = lax.axis_index("c")
    s = lax.axis_index("s")
    wid = s * NC + c
    pltpu.sync_copy(z_hbm.at[pl.ds(s * SEG, SEG)], acc.at[pl.ds(s * SEG, SEG)])
    plsc.subcore_barrier()

    def _step(j, carry):
        pltpu.sync_copy(src_hbm.at[wid, j], sidx)
        pltpu.sync_copy(dst_hbm.at[wid, j], didx)
        pltpu.async_copy(hs_hbm.at[sidx], rows, sem).wait()
        pltpu.sync_copy(rows, acc.at[didx], add=True)
        return carry

    lax.fori_loop(0, TPC, _step, 0)
    plsc.subcore_barrier()
    pltpu.sync_copy(acc.at[pl.ds(s * SEG, SEG)], out_hbm.at[c, pl.ds(s * SEG, SEG)])


_agg = pl.kernel(
    _agg_body,
    out_type=jax.ShapeDtypeStruct((NC, N_PAD, D), jnp.float32),
    mesh=_MESH,
    scratch_types=[
        pltpu.VMEM((CHUNK,), jnp.int32),
        pltpu.VMEM((CHUNK,), jnp.int32),
        pltpu.VMEM((CHUNK, D), jnp.float32),
        pltpu.VMEM_SHARED((N_PAD, D), jnp.float32),
        pltpu.SemaphoreType.DMA,
    ],
)


BLK = 1024
GRID = N_PAD // BLK


def _norm(a, b):
    return lax.rsqrt(jnp.maximum(a + b, 1.0))


def _mm_scale_body(x_ref, w_ref, d0_ref, d1_ref, o_ref):
    ns = _norm(d0_ref[...], d1_ref[...])
    o_ref[...] = jnp.dot(x_ref[...], w_ref[...], preferred_element_type=jnp.float32) * ns


def _post_mm_body(p_ref, w_ref, i0_ref, i1_ref, o0_ref, o1_ref, b_ref, o_ref):
    nin = _norm(i0_ref[...], i1_ref[...])
    h = jnp.maximum((p_ref[0] + p_ref[1]) * nin + b_ref[...], 0.0)
    nout = _norm(o0_ref[...], o1_ref[...])
    o_ref[...] = jnp.dot(h, w_ref[...], preferred_element_type=jnp.float32) * nout


def _post_body(p_ref, i0_ref, i1_ref, b_ref, o_ref):
    nin = _norm(i0_ref[...], i1_ref[...])
    o_ref[...] = jnp.maximum((p_ref[0] + p_ref[1]) * nin + b_ref[...], 0.0)


_col = pl.BlockSpec((BLK, 1), lambda i: (i, 0))
_full = pl.BlockSpec((D, D), lambda i: (0, 0))
_rowblk = pl.BlockSpec((BLK, D), lambda i: (i, 0))
_pblk = pl.BlockSpec((NC, BLK, D), lambda i: (0, i, 0))
_bias = pl.BlockSpec((1, D), lambda i: (0, 0))
_out_t = jax.ShapeDtypeStruct((N_PAD, D), jnp.float32)

_mm_scale = pl.pallas_call(
    _mm_scale_body,
    grid=(GRID,),
    in_specs=[_rowblk, _full, _col, _col],
    out_specs=_rowblk,
    out_shape=_out_t,
)

_post_mm = pl.pallas_call(
    _post_mm_body,
    grid=(GRID,),
    in_specs=[_pblk, _full, _col, _col, _col, _col, _bias],
    out_specs=_rowblk,
    out_shape=_out_t,
)

_post = pl.pallas_call(
    _post_body,
    grid=(GRID,),
    in_specs=[_pblk, _col, _col, _bias],
    out_specs=_rowblk,
    out_shape=_out_t,
)


def kernel(edge_index, x, W1, b1, W2, b2):
    src = edge_index[0]
    dst = edge_index[1]
    pad = N + (lax.iota(jnp.int32, E_PAD - E) % (N_PAD - N))
    src3 = jnp.concatenate([src, pad]).reshape(NW, TPC, CHUNK)
    dst3 = jnp.concatenate([dst, pad]).reshape(NW, TPC, CHUNK)
    xp = jnp.pad(x, ((0, N_PAD - N), (0, 0)))
    z2 = jnp.zeros((2, N_PAD), jnp.float32)
    zN = jnp.zeros((N_PAD, D), jnp.float32)

    degs = _deg(src3, dst3, z2)
    dout0 = degs[0, 0].reshape(N_PAD, 1)
    dout1 = degs[1, 0].reshape(N_PAD, 1)
    din0 = degs[0, 1].reshape(N_PAD, 1)
    din1 = degs[1, 1].reshape(N_PAD, 1)
    b1r = b1.reshape(1, D)
    b2r = b2.reshape(1, D)

    hs1 = _mm_scale(xp, W1, dout0, dout1)
    p1 = _agg(src3, dst3, hs1, zN)
    hs2 = _post_mm(p1, W2, din0, din1, dout0, dout1, b1r)
    p2 = _agg(src3, dst3, hs2, zN)
    out = _post(p2, din0, din1, b2r)
    return out[:N]


# speedup vs baseline: 9.3076x; 9.3076x over previous
import functools

import jax
import jax.numpy as jnp
from jax import lax
from jax.experimental import pallas as pl
from jax.experimental.pallas import tpu as pltpu
from jax.experimental.pallas import tpu_sc as plsc

N = 10000
D = 128
NC = 2
NS = 16
NW = NC * NS
CHUNK = 128
N_PAD = 10240
E = 320000
TPC = 79
EPT = TPC * CHUNK
E_PAD = NW * EPT
SEG = N_PAD // NS

@functools.cache
def _sc_kernels():
    mesh = plsc.VectorSubcoreMesh(
        core_axis_name="c", subcore_axis_name="s", num_cores=NC, num_subcores=NS
    )
    deg = pl.kernel(
        _deg_body,
        out_type=jax.ShapeDtypeStruct((NC, 2, N_PAD), jnp.float32),
        mesh=mesh,
        scratch_types=[
            pltpu.VMEM((CHUNK,), jnp.int32),
            pltpu.VMEM((CHUNK,), jnp.float32),
            pltpu.VMEM_SHARED((N_PAD,), jnp.float32),
            pltpu.VMEM_SHARED((N_PAD,), jnp.float32),
        ],
    )
    agg = pl.kernel(
        _agg_body,
        out_type=jax.ShapeDtypeStruct((NC, N_PAD, D), jnp.float32),
        mesh=mesh,
        scratch_types=[
            pltpu.VMEM((CHUNK,), jnp.int32),
            pltpu.VMEM((CHUNK,), jnp.int32),
            pltpu.VMEM((CHUNK, D), jnp.float32),
            pltpu.VMEM_SHARED((N_PAD, D), jnp.float32),
            pltpu.SemaphoreType.DMA,
        ],
    )
    return deg, agg


def _deg_body(src_hbm, dst_hbm, z2_hbm, out_hbm, idx_v, ones_v, dout_acc, din_acc):
    c = lax.axis_index("c")
    s = lax.axis_index("s")
    wid = s * NC + c
    pltpu.sync_copy(z2_hbm.at[0, pl.ds(s * SEG, SEG)], dout_acc.at[pl.ds(s * SEG, SEG)])
    pltpu.sync_copy(z2_hbm.at[1, pl.ds(s * SEG, SEG)], din_acc.at[pl.ds(s * SEG, SEG)])

    def _ones(i, carry):
        ones_v[pl.ds(i * 16, 16)] = jnp.ones((16,), jnp.float32)
        return carry

    lax.fori_loop(0, CHUNK // 16, _ones, 0)
    plsc.subcore_barrier()

    def _step(j, carry):
        pltpu.sync_copy(src_hbm.at[wid, j], idx_v)
        pltpu.sync_copy(ones_v, dout_acc.at[idx_v], add=True)
        pltpu.sync_copy(dst_hbm.at[wid, j], idx_v)
        pltpu.sync_copy(ones_v, din_acc.at[idx_v], add=True)
        return carry

    lax.fori_loop(0, TPC, _step, 0)
    plsc.subcore_barrier()
    pltpu.sync_copy(dout_acc.at[pl.ds(s * SEG, SEG)], out_hbm.at[c, 0, pl.ds(s * SEG, SEG)])
    pltpu.sync_copy(din_acc.at[pl.ds(s * SEG, SEG)], out_hbm.at[c, 1, pl.ds(s * SEG, SEG)])


def _agg_body(src_hbm, dst_hbm, hs_hbm, z_hbm, out_hbm, sidx, didx, rows, acc, sem):
    c = lax.axis_index("c")
    s = lax.axis_index("s")
    wid = s * NC + c
    pltpu.sync_copy(z_hbm.at[pl.ds(s * SEG, SEG)], acc.at[pl.ds(s * SEG, SEG)])
    plsc.subcore_barrier()

    def _step(j, carry):
        pltpu.sync_copy(src_hbm.at[wid, j], sidx)
        pltpu.sync_copy(dst_hbm.at[wid, j], didx)
        pltpu.async_copy(hs_hbm.at[sidx], rows, sem).wait()
        pltpu.sync_copy(rows, acc.at[didx], add=True)
        return carry

    lax.fori_loop(0, TPC, _step, 0)
    plsc.subcore_barrier()
    pltpu.sync_copy(acc.at[pl.ds(s * SEG, SEG)], out_hbm.at[c, pl.ds(s * SEG, SEG)])


BLK = 1024
GRID = N_PAD // BLK


def _norm(a, b):
    return lax.rsqrt(jnp.maximum(a + b, 1.0))


def _mm_scale_body(x_ref, w_ref, d0_ref, d1_ref, o_ref):
    ns = _norm(d0_ref[...], d1_ref[...])
    o_ref[...] = jnp.dot(x_ref[...], w_ref[...], preferred_element_type=jnp.float32) * ns


def _post_mm_body(p_ref, w_ref, i0_ref, i1_ref, o0_ref, o1_ref, b_ref, o_ref):
    nin = _norm(i0_ref[...], i1_ref[...])
    h = jnp.maximum((p_ref[0] + p_ref[1]) * nin + b_ref[...], 0.0)
    nout = _norm(o0_ref[...], o1_ref[...])
    o_ref[...] = jnp.dot(h, w_ref[...], preferred_element_type=jnp.float32) * nout


def _post_body(p_ref, i0_ref, i1_ref, b_ref, o_ref):
    nin = _norm(i0_ref[...], i1_ref[...])
    o_ref[...] = jnp.maximum((p_ref[0] + p_ref[1]) * nin + b_ref[...], 0.0)


_col = pl.BlockSpec((BLK, 1), lambda i: (i, 0))
_full = pl.BlockSpec((D, D), lambda i: (0, 0))
_rowblk = pl.BlockSpec((BLK, D), lambda i: (i, 0))
_pblk = pl.BlockSpec((NC, BLK, D), lambda i: (0, i, 0))
_bias = pl.BlockSpec((1, D), lambda i: (0, 0))
_out_t = jax.ShapeDtypeStruct((N_PAD, D), jnp.float32)

_mm_scale = pl.pallas_call(
    _mm_scale_body,
    grid=(GRID,),
    in_specs=[_rowblk, _full, _col, _col],
    out_specs=_rowblk,
    out_shape=_out_t,
)

_post_mm = pl.pallas_call(
    _post_mm_body,
    grid=(GRID,),
    in_specs=[_pblk, _full, _col, _col, _col, _col, _bias],
    out_specs=_rowblk,
    out_shape=_out_t,
)

_post = pl.pallas_call(
    _post_body,
    grid=(GRID,),
    in_specs=[_pblk, _col, _col, _bias],
    out_specs=_rowblk,
    out_shape=_out_t,
)


def kernel(edge_index, x, W1, b1, W2, b2):
    src = edge_index[0]
    dst = edge_index[1]
    pad = N + (lax.iota(jnp.int32, E_PAD - E) % (N_PAD - N))
    src3 = jnp.concatenate([src, pad]).reshape(NW, TPC, CHUNK)
    dst3 = jnp.concatenate([dst, pad]).reshape(NW, TPC, CHUNK)
    xp = jnp.pad(x, ((0, N_PAD - N), (0, 0)))
    z2 = jnp.zeros((2, N_PAD), jnp.float32)
    zN = jnp.zeros((N_PAD, D), jnp.float32)

    _deg, _agg = _sc_kernels()
    degs = _deg(src3, dst3, z2)
    dout0 = degs[0, 0].reshape(N_PAD, 1)
    dout1 = degs[1, 0].reshape(N_PAD, 1)
    din0 = degs[0, 1].reshape(N_PAD, 1)
    din1 = degs[1, 1].reshape(N_PAD, 1)
    b1r = b1.reshape(1, D)
    b2r = b2.reshape(1, D)

    hs1 = _mm_scale(xp, W1, dout0, dout1)
    p1 = _agg(src3, dst3, hs1, zN)
    hs2 = _post_mm(p1, W2, din0, din1, dout0, dout1, b1r)
    p2 = _agg(src3, dst3, hs2, zN)
    out = _post(p2, din0, din1, b2r)
    return out[:N]


# speedup vs baseline: 19.7841x; 2.1256x over previous
import functools

import jax
import jax.numpy as jnp
from jax import lax
from jax.experimental import pallas as pl
from jax.experimental.pallas import tpu as pltpu
from jax.experimental.pallas import tpu_sc as plsc

N = 10000
D = 128
NC = 2
NS = 16
NW = NC * NS
CHUNK = 128
N_PAD = 10240
E = 320000
TPC = 80
TPCA = TPC + 2
SEG = N_PAD // NS

@functools.cache
def _sc_kernels():
    mesh = plsc.VectorSubcoreMesh(
        core_axis_name="c", subcore_axis_name="s", num_cores=NC, num_subcores=NS
    )
    deg = pl.kernel(
        _deg_body,
        out_type=jax.ShapeDtypeStruct((NC, 2, N_PAD), jnp.float32),
        mesh=mesh,
        scratch_types=[
            pltpu.VMEM((TPCA, CHUNK), jnp.int32),
            pltpu.VMEM((TPCA, CHUNK), jnp.int32),
            pltpu.VMEM((CHUNK,), jnp.float32),
            pltpu.VMEM_SHARED((N_PAD,), jnp.float32),
            pltpu.VMEM_SHARED((N_PAD,), jnp.float32),
            pltpu.SemaphoreType.DMA,
        ],
    )
    agg = pl.kernel(
        _agg_body,
        out_type=jax.ShapeDtypeStruct((NC, N_PAD, D), jnp.float32),
        mesh=mesh,
        scratch_types=[
            pltpu.VMEM((CHUNK,), jnp.int32),
            pltpu.VMEM((CHUNK,), jnp.int32),
            pltpu.VMEM((TPC, CHUNK), jnp.int32),
            pltpu.VMEM((CHUNK, D), jnp.float32),
            pltpu.VMEM((CHUNK, D), jnp.float32),
            pltpu.VMEM_SHARED((N_PAD, D), jnp.float32),
            pltpu.SemaphoreType.DMA,
            pltpu.SemaphoreType.DMA,
            pltpu.SemaphoreType.DMA,
            pltpu.SemaphoreType.DMA,
            pltpu.SemaphoreType.DMA,
            pltpu.SemaphoreType.DMA,
        ],
    )
    return deg, agg


def _deg_body(src_hbm, dst_hbm, z2_hbm, out_hbm, sidx, didx, ones_v, dout_acc, din_acc, dsem):
    c = lax.axis_index("c")
    s = lax.axis_index("s")
    wid = s * NC + c
    pltpu.sync_copy(z2_hbm.at[0, pl.ds(s * SEG, SEG)], dout_acc.at[pl.ds(s * SEG, SEG)])
    pltpu.sync_copy(z2_hbm.at[1, pl.ds(s * SEG, SEG)], din_acc.at[pl.ds(s * SEG, SEG)])
    pltpu.sync_copy(src_hbm.at[wid], sidx)
    pltpu.sync_copy(dst_hbm.at[wid], didx)

    def _ones(i, carry):
        ones_v[pl.ds(i * 16, 16)] = jnp.ones((16,), jnp.float32)
        return carry

    lax.fori_loop(0, CHUNK // 16, _ones, 0)
    plsc.subcore_barrier()

    def _step(j, carry):
        pltpu.async_copy(ones_v, dout_acc.at[sidx.at[j]], dsem, add=True)
        pltpu.async_copy(ones_v, din_acc.at[didx.at[j]], dsem, add=True)
        return carry

    lax.fori_loop(0, TPC, _step, 0)

    def _drain(j, carry):
        pltpu.make_async_copy(ones_v, dout_acc.at[sidx.at[0]], dsem).wait()
        pltpu.make_async_copy(ones_v, din_acc.at[didx.at[0]], dsem).wait()
        return carry

    lax.fori_loop(0, TPC, _drain, 0)
    plsc.subcore_barrier()
    pltpu.sync_copy(dout_acc.at[pl.ds(s * SEG, SEG)], out_hbm.at[c, 0, pl.ds(s * SEG, SEG)])
    pltpu.sync_copy(din_acc.at[pl.ds(s * SEG, SEG)], out_hbm.at[c, 1, pl.ds(s * SEG, SEG)])


def _agg_body(src_hbm, dst_hbm, hs_hbm, z_hbm, out_hbm,
              sr0, sr1, didx, rows0, rows1, acc, g0, g1, s0, s1, i0, i1):
    c = lax.axis_index("c")
    s = lax.axis_index("s")
    wid = s * NC + c
    pltpu.sync_copy(z_hbm.at[pl.ds(s * SEG, SEG)], acc.at[pl.ds(s * SEG, SEG)])
    pltpu.sync_copy(dst_hbm.at[wid, pl.ds(0, TPC)], didx)
    plsc.subcore_barrier()

    def ifetch(j, buf, sem):
        pltpu.async_copy(src_hbm.at[wid, j], buf, sem)

    def iwait(buf, sem):
        pltpu.make_async_copy(src_hbm.at[wid, 0], buf, sem).wait()

    def gather(sbuf, buf, sem):
        pltpu.async_copy(hs_hbm.at[sbuf], buf, sem)

    def gwait(buf, sem):
        pltpu.make_async_copy(hs_hbm.at[sr0], buf, sem).wait()

    def scat(j, buf, sem):
        pltpu.async_copy(buf, acc.at[didx.at[j]], sem, add=True)

    def swait(buf, sem):
        pltpu.make_async_copy(buf, acc.at[didx.at[0]], sem).wait()

    ifetch(0, sr0, i0)
    ifetch(1, sr1, i1)
    iwait(sr0, i0)
    gather(sr0, rows0, g0)

    def _body(i, carry):
        j0 = 2 * i

        @pl.when(i > 0)
        def _():
            swait(rows1, s1)

        iwait(sr1, i1)
        gather(sr1, rows1, g1)
        gwait(rows0, g0)
        ifetch(j0 + 2, sr0, i0)
        scat(j0, rows0, s0)
        swait(rows0, s0)
        iwait(sr0, i0)
        gather(sr0, rows0, g0)
        gwait(rows1, g1)
        ifetch(j0 + 3, sr1, i1)
        scat(j0 + 1, rows1, s1)
        return carry

    lax.fori_loop(0, TPC // 2, _body, 0)
    gwait(rows0, g0)
    swait(rows1, s1)
    iwait(sr1, i1)
    plsc.subcore_barrier()
    pltpu.sync_copy(acc.at[pl.ds(s * SEG, SEG)], out_hbm.at[c, pl.ds(s * SEG, SEG)])


BLK = 1024
GRID = N_PAD // BLK


def _norm(a, b):
    return lax.rsqrt(jnp.maximum(a + b, 1.0))


def _mm_scale_body(x_ref, w_ref, d0_ref, d1_ref, o_ref):
    ns = _norm(d0_ref[...], d1_ref[...])
    o_ref[...] = jnp.dot(x_ref[...], w_ref[...], preferred_element_type=jnp.float32) * ns


def _post_mm_body(p_ref, w_ref, i0_ref, i1_ref, o0_ref, o1_ref, b_ref, o_ref):
    nin = _norm(i0_ref[...], i1_ref[...])
    h = jnp.maximum((p_ref[0] + p_ref[1]) * nin + b_ref[...], 0.0)
    nout = _norm(o0_ref[...], o1_ref[...])
    o_ref[...] = jnp.dot(h, w_ref[...], preferred_element_type=jnp.float32) * nout


def _post_body(p_ref, i0_ref, i1_ref, b_ref, o_ref):
    nin = _norm(i0_ref[...], i1_ref[...])
    o_ref[...] = jnp.maximum((p_ref[0] + p_ref[1]) * nin + b_ref[...], 0.0)


_col = pl.BlockSpec((BLK, 1), lambda i: (i, 0))
_full = pl.BlockSpec((D, D), lambda i: (0, 0))
_rowblk = pl.BlockSpec((BLK, D), lambda i: (i, 0))
_pblk = pl.BlockSpec((NC, BLK, D), lambda i: (0, i, 0))
_bias = pl.BlockSpec((1, D), lambda i: (0, 0))
_out_t = jax.ShapeDtypeStruct((N_PAD, D), jnp.float32)

_mm_scale = pl.pallas_call(
    _mm_scale_body,
    grid=(GRID,),
    in_specs=[_rowblk, _full, _col, _col],
    out_specs=_rowblk,
    out_shape=_out_t,
)

_post_mm = pl.pallas_call(
    _post_mm_body,
    grid=(GRID,),
    in_specs=[_pblk, _full, _col, _col, _col, _col, _bias],
    out_specs=_rowblk,
    out_shape=_out_t,
)

_post = pl.pallas_call(
    _post_body,
    grid=(GRID,),
    in_specs=[_pblk, _col, _col, _bias],
    out_specs=_rowblk,
    out_shape=_out_t,
)


def kernel(edge_index, x, W1, b1, W2, b2):
    src = edge_index[0]
    dst = edge_index[1]
    pad = N + (lax.iota(jnp.int32, NW * TPC * CHUNK - E) % (N_PAD - N))
    main = jnp.concatenate([src, pad]).reshape(NW, TPC, CHUNK)
    maind = jnp.concatenate([dst, pad]).reshape(NW, TPC, CHUNK)
    padc = (N + (lax.iota(jnp.int32, NW * 2 * CHUNK) % (N_PAD - N))).reshape(NW, 2, CHUNK)
    src3 = jnp.concatenate([main, padc], axis=1)
    dst3 = jnp.concatenate([maind, padc], axis=1)
    xp = jnp.pad(x, ((0, N_PAD - N), (0, 0)))
    z2 = jnp.zeros((2, N_PAD), jnp.float32)
    zN = jnp.zeros((N_PAD, D), jnp.float32)

    _deg, _agg = _sc_kernels()
    degs = _deg(src3, dst3, z2)
    dout0 = degs[0, 0].reshape(N_PAD, 1)
    dout1 = degs[1, 0].reshape(N_PAD, 1)
    din0 = degs[0, 1].reshape(N_PAD, 1)
    din1 = degs[1, 1].reshape(N_PAD, 1)
    b1r = b1.reshape(1, D)
    b2r = b2.reshape(1, D)

    hs1 = _mm_scale(xp, W1, dout0, dout1)
    p1 = _agg(src3, dst3, hs1, zN)
    hs2 = _post_mm(p1, W2, din0, din1, dout0, dout1, b1r)
    p2 = _agg(src3, dst3, hs2, zN)
    out = _post(p2, din0, din1, b2r)
    return out[:N]
